# Initial kernel scaffold; baseline (speedup 1.0000x reference)
import jax
import jax.numpy as jnp
from jax.experimental import pallas as pl


def kernel(g1, x1, g2, x2, g3, x3, W, b):
    raise NotImplementedError("write your pallas kernel here")


# speedup vs baseline: 6.0561x; 6.0561x over previous
import functools

import jax
import jax.numpy as jnp
from jax import lax
from jax.experimental import pallas as pl
from jax.experimental.pallas import tpu as pltpu
from jax.experimental.pallas import tpu_sc as plsc

N = 10000
E = 320000
D = 128

NC = 2
NS = 16
NW = NC * NS

WIN = 80
MW = E // (NW * WIN)
DTOT = 6 * E
DPT = DTOT // NW
DC = 25
DCH = DPT // (WIN * DC)

ROWS_PER_SUB_DEG = 6 * N // NS
ROWS_PER_SUB_AGG = N // NS

_MESH = plsc.VectorSubcoreMesh(core_axis_name="c", subcore_axis_name="s")
_SC_PARAMS = pltpu.CompilerParams(use_tc_tiling_on_sc=False)


def _deg_call(deg_idx, ones_hbm, zeros_hbm):
  @functools.partial(
      pl.kernel,
      out_type=jax.ShapeDtypeStruct((NC, NS, ROWS_PER_SUB_DEG, 16),
                                    jnp.float32),
      mesh=_MESH,
      scratch_types=[
          pltpu.VMEM((DC, WIN), jnp.int32),
          pltpu.VMEM((WIN, 16), jnp.float32),
          pltpu.VMEM_SHARED((6 * N, 16), jnp.float32),
      ],
      compiler_params=_SC_PARAMS,
  )
  def k(idx_hbm, ones_h, zeros_h, out_hbm, idx_v, ones_v, table_sh):
    cid = lax.axis_index("c")
    sid = lax.axis_index("s")
    wid = sid * NC + cid
    pltpu.sync_copy(ones_h, ones_v)
    pltpu.sync_copy(zeros_h, table_sh.at[pl.ds(sid * ROWS_PER_SUB_DEG,
                                               ROWS_PER_SUB_DEG)])
    plsc.subcore_barrier()

    @pl.loop(0, DCH)
    def _chunk(ch):
      pltpu.sync_copy(idx_hbm.at[wid, ch], idx_v)

      @pl.loop(0, DC)
      def _win(w):
        pltpu.sync_copy(ones_v, table_sh.at[idx_v.at[w]], add=True)

    plsc.subcore_barrier()
    pltpu.sync_copy(
        table_sh.at[pl.ds(sid * ROWS_PER_SUB_DEG, ROWS_PER_SUB_DEG)],
        out_hbm.at[cid, sid])

  return jnp.reshape(k(deg_idx, ones_hbm, zeros_hbm), (NC, 6 * N, 16))


def _h_call(xs, degp):
  bn = 2000
  nb = N // bn

  def body(deg_ref, x_ref, h_ref):
    deg = (deg_ref[0] + deg_ref[1])[:, 0:1]
    norm = jnp.where(deg > 0.0, lax.rsqrt(jnp.maximum(deg, 1.0)), 0.0)
    h_ref[0] = x_ref[0] * norm

  return pl.pallas_call(
      body,
      grid=(3, nb),
      in_specs=[
          pl.BlockSpec((NC, bn, 16), lambda k, j: (0, 2 * k * nb + j, 0)),
          pl.BlockSpec((1, bn, D), lambda k, j: (k, j, 0)),
      ],
      out_specs=pl.BlockSpec((1, bn, D), lambda k, j: (k, j, 0)),
      out_shape=jax.ShapeDtypeStruct((3, N, D), jnp.float32),
  )(degp, xs)


def _msg_call(h, src_idx, dst_idx, zeros_hbm):
  @functools.partial(
      pl.kernel,
      out_type=jax.ShapeDtypeStruct((3, NC, NS, ROWS_PER_SUB_AGG, D),
                                    jnp.float32),
      mesh=_MESH,
      scratch_types=[
          pltpu.VMEM((MW, WIN), jnp.int32),
          pltpu.VMEM((MW, WIN), jnp.int32),
          pltpu.VMEM((WIN, D), jnp.float32),
          pltpu.VMEM_SHARED((N, D), jnp.float32),
      ],
      compiler_params=_SC_PARAMS,
  )
  def k(h_hbm, sidx_hbm, didx_hbm, zeros_h, out_hbm, src_v, dst_v, rows_v,
        agg_sh):
    cid = lax.axis_index("c")
    sid = lax.axis_index("s")
    wid = sid * NC + cid
    my_rows = pl.ds(sid * ROWS_PER_SUB_AGG, ROWS_PER_SUB_AGG)

    for g in range(3):
      pltpu.sync_copy(zeros_h, agg_sh.at[my_rows])
      pltpu.sync_copy(sidx_hbm.at[g, wid], src_v)
      pltpu.sync_copy(didx_hbm.at[g, wid], dst_v)
      plsc.subcore_barrier()

      @pl.loop(0, MW)
      def _win(w):
        pltpu.sync_copy(h_hbm.at[g].at[src_v.at[w]], rows_v)
        pltpu.sync_copy(rows_v, agg_sh.at[dst_v.at[w]], add=True)

      plsc.subcore_barrier()
      pltpu.sync_copy(agg_sh.at[my_rows], out_hbm.at[g, cid, sid])
      plsc.subcore_barrier()

  return jnp.reshape(k(h, src_idx, dst_idx, zeros_hbm), (3, NC, N, D))


def _head_call(aggp, degp, W, b):
  bn = 2000
  nb = N // bn
  scale = 1.0 / (3.0 * N * D)

  def body(agg_ref, deg_ref, w_ref, b_ref, out_ref):
    g = pl.program_id(0)
    j = pl.program_id(1)
    deg = (deg_ref[0] + deg_ref[1])[:, 0:1]
    norm = jnp.where(deg > 0.0, lax.rsqrt(jnp.maximum(deg, 1.0)), 0.0)
    agg = (agg_ref[0, 0] + agg_ref[0, 1]) * norm
    y = lax.dot_general(agg, w_ref[...], (((1,), (0,)), ((), ())),
                        preferred_element_type=jnp.float32,
                        precision=lax.Precision.HIGHEST)
    y = jnp.maximum(y + b_ref[...], 0.0)

    @pl.when((g == 0) & (j == 0))
    def _():
      out_ref[...] = jnp.zeros((1, 1), jnp.float32)

    out_ref[...] += jnp.reshape(jnp.sum(y) * scale, (1, 1))

  return pl.pallas_call(
      body,
      grid=(3, nb),
      in_specs=[
          pl.BlockSpec((1, NC, bn, D), lambda k, j: (k, 0, j, 0)),
          pl.BlockSpec((NC, bn, 16), lambda k, j: (0, (2 * k + 1) * nb + j, 0)),
          pl.BlockSpec((D, D), lambda k, j: (0, 0)),
          pl.BlockSpec((1, D), lambda k, j: (0, 0)),
      ],
      out_specs=pl.BlockSpec((1, 1), lambda k, j: (0, 0)),
      out_shape=jax.ShapeDtypeStruct((1, 1), jnp.float32),
  )(aggp, degp, W, b)


@jax.jit
def kernel(g1, x1, g2, x2, g3, x3, W, b):
  xs = jnp.stack([x1, x2, x3])
  deg_idx = jnp.concatenate([
      g1[0], g1[1] + N,
      g2[0] + 2 * N, g2[1] + 3 * N,
      g3[0] + 4 * N, g3[1] + 5 * N,
  ]).reshape(NW, DCH, DC, WIN)
  src_idx = jnp.stack([g1[0], g2[0], g3[0]]).reshape(3, NW, MW, WIN)
  dst_idx = jnp.stack([g1[1], g2[1], g3[1]]).reshape(3, NW, MW, WIN)
  ones16 = jnp.ones((WIN, 16), jnp.float32)
  zeros_deg = jnp.zeros((ROWS_PER_SUB_DEG, 16), jnp.float32)
  zeros_agg = jnp.zeros((ROWS_PER_SUB_AGG, D), jnp.float32)

  degp = _deg_call(deg_idx, ones16, zeros_deg)
  h = _h_call(xs, degp)
  aggp = _msg_call(h, src_idx, dst_idx, zeros_agg)
  out = _head_call(aggp, degp, W, jnp.reshape(b, (1, D)))
  return out[0, 0]


# speedup vs baseline: 7.1634x; 1.1828x over previous
import functools

import jax
import jax.numpy as jnp
from jax import lax
from jax.experimental import pallas as pl
from jax.experimental.pallas import tpu as pltpu
from jax.experimental.pallas import tpu_sc as plsc

N = 10000
E = 320000
D = 128

NC = 2
NS = 16
NW = NC * NS

WIN = 80
MW = E // (NW * WIN)
DTOT = 6 * E
DPT = DTOT // NW
DC = 25
DCH = DPT // (WIN * DC)

ROWS_PER_SUB_DEG = 6 * N // NS
ROWS_PER_SUB_AGG = N // NS

_MESH = plsc.VectorSubcoreMesh(core_axis_name="c", subcore_axis_name="s")
_SC_PARAMS = pltpu.CompilerParams(use_tc_tiling_on_sc=False)


def _deg_call(deg_idx, ones_hbm, zeros_hbm):
  @functools.partial(
      pl.kernel,
      out_type=jax.ShapeDtypeStruct((NC, NS, ROWS_PER_SUB_DEG, 16),
                                    jnp.float32),
      mesh=_MESH,
      scratch_types=[
          pltpu.VMEM((DC, WIN), jnp.int32),
          pltpu.VMEM((WIN, 16), jnp.float32),
          pltpu.VMEM_SHARED((6 * N, 16), jnp.float32),
      ],
      compiler_params=_SC_PARAMS,
  )
  def k(idx_hbm, ones_h, zeros_h, out_hbm, idx_v, ones_v, table_sh):
    cid = lax.axis_index("c")
    sid = lax.axis_index("s")
    wid = sid * NC + cid
    pltpu.sync_copy(ones_h, ones_v)
    pltpu.sync_copy(zeros_h, table_sh.at[pl.ds(sid * ROWS_PER_SUB_DEG,
                                               ROWS_PER_SUB_DEG)])
    plsc.subcore_barrier()

    @pl.loop(0, DCH)
    def _chunk(ch):
      pltpu.sync_copy(idx_hbm.at[wid, ch], idx_v)

      @pl.loop(0, DC)
      def _win(w):
        pltpu.sync_copy(ones_v, table_sh.at[idx_v.at[w]], add=True)

    plsc.subcore_barrier()
    pltpu.sync_copy(
        table_sh.at[pl.ds(sid * ROWS_PER_SUB_DEG, ROWS_PER_SUB_DEG)],
        out_hbm.at[cid, sid])

  return jnp.reshape(k(deg_idx, ones_hbm, zeros_hbm), (NC, 6 * N, 16))


def _h_call(xs, degp):
  bn = 2000
  nb = N // bn

  def body(deg_ref, x_ref, h_ref):
    deg = (deg_ref[0] + deg_ref[1])[:, 0:1]
    norm = jnp.where(deg > 0.0, lax.rsqrt(jnp.maximum(deg, 1.0)), 0.0)
    h_ref[0] = x_ref[0] * norm

  return pl.pallas_call(
      body,
      grid=(3, nb),
      in_specs=[
          pl.BlockSpec((NC, bn, 16), lambda k, j: (0, 2 * k * nb + j, 0)),
          pl.BlockSpec((1, bn, D), lambda k, j: (k, j, 0)),
      ],
      out_specs=pl.BlockSpec((1, bn, D), lambda k, j: (k, j, 0)),
      out_shape=jax.ShapeDtypeStruct((3, N, D), jnp.float32),
  )(degp, xs)


def _msg_call(h, src_idx, dst_idx, zeros_hbm):
  @functools.partial(
      pl.kernel,
      out_type=jax.ShapeDtypeStruct((3, NC, NS, ROWS_PER_SUB_AGG, D),
                                    jnp.float32),
      mesh=_MESH,
      scratch_types=[
          pltpu.VMEM((MW, WIN), jnp.int32),
          pltpu.VMEM((MW, WIN), jnp.int32),
          pltpu.VMEM((WIN, D), jnp.float32),
          pltpu.VMEM((WIN, D), jnp.float32),
          pltpu.VMEM_SHARED((N, D), jnp.float32),
          pltpu.SemaphoreType.DMA,
          pltpu.SemaphoreType.DMA,
      ],
      compiler_params=_SC_PARAMS,
  )
  def k(h_hbm, sidx_hbm, didx_hbm, zeros_h, out_hbm, src_v, dst_v, rows_a,
        rows_b, agg_sh, sem_a, sem_b):
    cid = lax.axis_index("c")
    sid = lax.axis_index("s")
    wid = sid * NC + cid
    my_rows = pl.ds(sid * ROWS_PER_SUB_AGG, ROWS_PER_SUB_AGG)

    for g in range(3):
      pltpu.sync_copy(zeros_h, agg_sh.at[my_rows])
      pltpu.sync_copy(sidx_hbm.at[g, wid], src_v)
      pltpu.sync_copy(didx_hbm.at[g, wid], dst_v)
      plsc.subcore_barrier()

      pltpu.async_copy(h_hbm.at[g].at[src_v.at[0]], rows_a, sem_a)

      @pl.loop(0, MW // 2)
      def _pair(i):
        w0 = 2 * i
        pltpu.make_async_copy(h_hbm.at[g].at[src_v.at[w0]], rows_a,
                              sem_a).wait()
        pltpu.async_copy(h_hbm.at[g].at[src_v.at[w0 + 1]], rows_b, sem_b)
        pltpu.sync_copy(rows_a, agg_sh.at[dst_v.at[w0]], add=True)
        pltpu.make_async_copy(h_hbm.at[g].at[src_v.at[w0 + 1]], rows_b,
                              sem_b).wait()
        pltpu.async_copy(h_hbm.at[g].at[src_v.at[w0 + 2]], rows_a, sem_a)
        pltpu.sync_copy(rows_b, agg_sh.at[dst_v.at[w0 + 1]], add=True)

      pltpu.make_async_copy(h_hbm.at[g].at[src_v.at[MW - 1]], rows_a,
                            sem_a).wait()
      pltpu.sync_copy(rows_a, agg_sh.at[dst_v.at[MW - 1]], add=True)

      plsc.subcore_barrier()
      pltpu.sync_copy(agg_sh.at[my_rows], out_hbm.at[g, cid, sid])
      plsc.subcore_barrier()

  return jnp.reshape(k(h, src_idx, dst_idx, zeros_hbm), (3, NC, N, D))


def _head_call(aggp, degp, W, b):
  bn = 2000
  nb = N // bn
  scale = 1.0 / (3.0 * N * D)

  def body(agg_ref, deg_ref, w_ref, b_ref, out_ref):
    g = pl.program_id(0)
    j = pl.program_id(1)
    deg = (deg_ref[0] + deg_ref[1])[:, 0:1]
    norm = jnp.where(deg > 0.0, lax.rsqrt(jnp.maximum(deg, 1.0)), 0.0)
    agg = (agg_ref[0, 0] + agg_ref[0, 1]) * norm
    y = lax.dot_general(agg, w_ref[...], (((1,), (0,)), ((), ())),
                        preferred_element_type=jnp.float32,
                        precision=lax.Precision.HIGHEST)
    y = jnp.maximum(y + b_ref[...], 0.0)

    @pl.when((g == 0) & (j == 0))
    def _():
      out_ref[...] = jnp.zeros((1, 1), jnp.float32)

    out_ref[...] += jnp.reshape(jnp.sum(y) * scale, (1, 1))

  return pl.pallas_call(
      body,
      grid=(3, nb),
      in_specs=[
          pl.BlockSpec((1, NC, bn, D), lambda k, j: (k, 0, j, 0)),
          pl.BlockSpec((NC, bn, 16), lambda k, j: (0, (2 * k + 1) * nb + j, 0)),
          pl.BlockSpec((D, D), lambda k, j: (0, 0)),
          pl.BlockSpec((1, D), lambda k, j: (0, 0)),
      ],
      out_specs=pl.BlockSpec((1, 1), lambda k, j: (0, 0)),
      out_shape=jax.ShapeDtypeStruct((1, 1), jnp.float32),
  )(aggp, degp, W, b)


@jax.jit
def kernel(g1, x1, g2, x2, g3, x3, W, b):
  xs = jnp.stack([x1, x2, x3])
  deg_idx = jnp.concatenate([
      g1[0], g1[1] + N,
      g2[0] + 2 * N, g2[1] + 3 * N,
      g3[0] + 4 * N, g3[1] + 5 * N,
  ]).reshape(NW, DCH, DC, WIN)
  src_idx = jnp.stack([g1[0], g2[0], g3[0]]).reshape(3, NW, MW, WIN)
  dst_idx = jnp.stack([g1[1], g2[1], g3[1]]).reshape(3, NW, MW, WIN)
  ones16 = jnp.ones((WIN, 16), jnp.float32)
  zeros_deg = jnp.zeros((ROWS_PER_SUB_DEG, 16), jnp.float32)
  zeros_agg = jnp.zeros((ROWS_PER_SUB_AGG, D), jnp.float32)

  degp = _deg_call(deg_idx, ones16, zeros_deg)
  h = _h_call(xs, degp)
  aggp = _msg_call(h, src_idx, dst_idx, zeros_agg)
  out = _head_call(aggp, degp, W, jnp.reshape(b, (1, D)))
  return out[0, 0]


# speedup vs baseline: 7.3832x; 1.0307x over previous
import functools

import jax
import jax.numpy as jnp
from jax import lax
from jax.experimental import pallas as pl
from jax.experimental.pallas import tpu as pltpu
from jax.experimental.pallas import tpu_sc as plsc

N = 10000
E = 320000
D = 128

NC = 2
NS = 16
NW = NC * NS

WIN = 80
MW = E // (NW * WIN)
DPT = 3 * E // NW
DC = 25
DCH = DPT // (WIN * DC)

ROWS_PER_SUB_DEG = 3 * N // NS
ROWS_PER_SUB_AGG = N // NS

_MESH = plsc.VectorSubcoreMesh(core_axis_name="c", subcore_axis_name="s")
_SC_PARAMS = pltpu.CompilerParams(use_tc_tiling_on_sc=False)


def _deg_call(deg_idx, ones_hbm, zeros_hbm):
  @functools.partial(
      pl.kernel,
      out_type=jax.ShapeDtypeStruct((NC, 2, NS, ROWS_PER_SUB_DEG, 16),
                                    jnp.float32),
      mesh=_MESH,
      scratch_types=[
          pltpu.VMEM((DC, WIN), jnp.int32),
          pltpu.VMEM((WIN, 16), jnp.float32),
          pltpu.VMEM_SHARED((3 * N, 16), jnp.float32),
      ],
      compiler_params=_SC_PARAMS,
  )
  def k(idx_hbm, ones_h, zeros_h, out_hbm, idx_v, ones_v, table_sh):
    cid = lax.axis_index("c")
    sid = lax.axis_index("s")
    wid = sid * NC + cid
    my_rows = pl.ds(sid * ROWS_PER_SUB_DEG, ROWS_PER_SUB_DEG)
    pltpu.sync_copy(ones_h, ones_v)

    for half in range(2):
      pltpu.sync_copy(zeros_h, table_sh.at[my_rows])
      plsc.subcore_barrier()

      @pl.loop(0, DCH)
      def _chunk(ch):
        pltpu.sync_copy(idx_hbm.at[half, wid, ch], idx_v)

        @pl.loop(0, DC)
        def _win(w):
          pltpu.sync_copy(ones_v, table_sh.at[idx_v.at[w]], add=True)

      plsc.subcore_barrier()
      pltpu.sync_copy(table_sh.at[my_rows], out_hbm.at[cid, half, sid])

  return jnp.reshape(k(deg_idx, ones_hbm, zeros_hbm), (NC, 6 * N, 16))


def _h_call(xs, degp):
  bn = 2000
  nb = N // bn

  def body(deg_ref, x_ref, h_ref):
    deg = (deg_ref[0] + deg_ref[1])[:, 0:1]
    norm = jnp.where(deg > 0.0, lax.rsqrt(jnp.maximum(deg, 1.0)), 0.0)
    h_ref[0] = x_ref[0] * norm

  return pl.pallas_call(
      body,
      grid=(3, nb),
      in_specs=[
          pl.BlockSpec((NC, bn, 16), lambda k, j: (0, 2 * k * nb + j, 0)),
          pl.BlockSpec((1, bn, D), lambda k, j: (k, j, 0)),
      ],
      out_specs=pl.BlockSpec((1, bn, D), lambda k, j: (k, j, 0)),
      out_shape=jax.ShapeDtypeStruct((3, N, D), jnp.float32),
  )(degp, xs)


def _msg_call(h, src_idx, dst_idx, zeros_hbm):
  @functools.partial(
      pl.kernel,
      out_type=jax.ShapeDtypeStruct((3, NC, NS, ROWS_PER_SUB_AGG, D),
                                    jnp.float32),
      mesh=_MESH,
      scratch_types=[
          pltpu.VMEM((MW, WIN), jnp.int32),
          pltpu.VMEM((MW, WIN), jnp.int32),
          pltpu.VMEM((WIN, D), jnp.float32),
          pltpu.VMEM((WIN, D), jnp.float32),
          pltpu.VMEM((WIN, D), jnp.float32),
          pltpu.VMEM_SHARED((N, D), jnp.float32),
          pltpu.SemaphoreType.DMA,
          pltpu.SemaphoreType.DMA,
          pltpu.SemaphoreType.DMA,
      ],
      compiler_params=_SC_PARAMS,
  )
  def k(h_hbm, sidx_hbm, didx_hbm, zeros_h, out_hbm, src_v, dst_v, rows_0,
        rows_1, rows_2, agg_sh, sem_0, sem_1, sem_2):
    cid = lax.axis_index("c")
    sid = lax.axis_index("s")
    wid = sid * NC + cid
    my_rows = pl.ds(sid * ROWS_PER_SUB_AGG, ROWS_PER_SUB_AGG)
    rows = [rows_0, rows_1, rows_2]
    sems = [sem_0, sem_1, sem_2]
    NBUF = 3
    NFULL = MW // NBUF

    for g in range(3):
      pltpu.sync_copy(zeros_h, agg_sh.at[my_rows])
      pltpu.sync_copy(sidx_hbm.at[g, wid], src_v)
      pltpu.sync_copy(didx_hbm.at[g, wid], dst_v)
      plsc.subcore_barrier()

      for j in range(NBUF):
        pltpu.sync_copy(h_hbm.at[g].at[src_v.at[j]], rows[j])
        pltpu.async_copy(rows[j], agg_sh.at[dst_v.at[j]], sems[j], add=True)

      @pl.loop(1, NFULL)
      def _blk(i):
        for j in range(NBUF):
          w = NBUF * i + j
          pltpu.make_async_copy(rows[j], agg_sh.at[dst_v.at[w - NBUF]],
                                sems[j]).wait()
          pltpu.sync_copy(h_hbm.at[g].at[src_v.at[w]], rows[j])
          pltpu.async_copy(rows[j], agg_sh.at[dst_v.at[w]], sems[j], add=True)

      last = NBUF * (NFULL - 1)
      tail_w = [NBUF * NFULL, NBUF * NFULL + 1]
      for j, w in enumerate(tail_w):
        pltpu.make_async_copy(rows[j], agg_sh.at[dst_v.at[last + j]],
                              sems[j]).wait()
        pltpu.sync_copy(h_hbm.at[g].at[src_v.at[w]], rows[j])
        pltpu.async_copy(rows[j], agg_sh.at[dst_v.at[w]], sems[j], add=True)
      for j, w in enumerate(tail_w):
        pltpu.make_async_copy(rows[j], agg_sh.at[dst_v.at[w]], sems[j]).wait()
      pltpu.make_async_copy(rows[2], agg_sh.at[dst_v.at[last + 2]],
                            sems[2]).wait()

      plsc.subcore_barrier()
      pltpu.sync_copy(agg_sh.at[my_rows], out_hbm.at[g, cid, sid])
      plsc.subcore_barrier()

  return jnp.reshape(k(h, src_idx, dst_idx, zeros_hbm), (3, NC, N, D))


def _head_call(aggp, degp, W, b):
  bn = 2000
  nb = N // bn
  scale = 1.0 / (3.0 * N * D)

  def body(agg_ref, deg_ref, w_ref, b_ref, out_ref):
    g = pl.program_id(0)
    j = pl.program_id(1)
    deg = (deg_ref[0] + deg_ref[1])[:, 0:1]
    norm = jnp.where(deg > 0.0, lax.rsqrt(jnp.maximum(deg, 1.0)), 0.0)
    agg = (agg_ref[0, 0] + agg_ref[0, 1]) * norm
    y = lax.dot_general(agg, w_ref[...], (((1,), (0,)), ((), ())),
                        preferred_element_type=jnp.float32,
                        precision=lax.Precision.HIGHEST)
    y = jnp.maximum(y + b_ref[...], 0.0)

    @pl.when((g == 0) & (j == 0))
    def _():
      out_ref[...] = jnp.zeros((1, 1), jnp.float32)

    out_ref[...] += jnp.reshape(jnp.sum(y) * scale, (1, 1))

  return pl.pallas_call(
      body,
      grid=(3, nb),
      in_specs=[
          pl.BlockSpec((1, NC, bn, D), lambda k, j: (k, 0, j, 0)),
          pl.BlockSpec((NC, bn, 16), lambda k, j: (0, (2 * k + 1) * nb + j, 0)),
          pl.BlockSpec((D, D), lambda k, j: (0, 0)),
          pl.BlockSpec((1, D), lambda k, j: (0, 0)),
      ],
      out_specs=pl.BlockSpec((1, 1), lambda k, j: (0, 0)),
      out_shape=jax.ShapeDtypeStruct((1, 1), jnp.float32),
  )(aggp, degp, W, b)


@jax.jit
def kernel(g1, x1, g2, x2, g3, x3, W, b):
  xs = jnp.stack([x1, x2, x3])
  deg_idx = jnp.stack([
      jnp.concatenate([g1[0], g1[1] + N, g2[0] + 2 * N]),
      jnp.concatenate([g2[1], g3[0] + N, g3[1] + 2 * N]),
  ]).reshape(2, NW, DCH, DC, WIN)
  src_idx = jnp.stack([g1[0], g2[0], g3[0]]).reshape(3, NW, MW, WIN)
  dst_idx = jnp.stack([g1[1], g2[1], g3[1]]).reshape(3, NW, MW, WIN)
  ones16 = jnp.ones((WIN, 16), jnp.float32)
  zeros_deg = jnp.zeros((ROWS_PER_SUB_DEG, 16), jnp.float32)
  zeros_agg = jnp.zeros((ROWS_PER_SUB_AGG, D), jnp.float32)

  degp = _deg_call(deg_idx, ones16, zeros_deg)
  h = _h_call(xs, degp)
  aggp = _msg_call(h, src_idx, dst_idx, zeros_agg)
  out = _head_call(aggp, degp, W, jnp.reshape(b, (1, D)))
  return out[0, 0]


# speedup vs baseline: 9.3914x; 1.2720x over previous
import functools

import jax
import jax.numpy as jnp
from jax import lax
from jax.experimental import pallas as pl
from jax.experimental.pallas import tpu as pltpu
from jax.experimental.pallas import tpu_sc as plsc

N = 10000
E = 320000
D = 128

NC = 2
NS = 16
NW = NC * NS

WIN = 80
MW = E // (NW * WIN)
DPT = 3 * E // NW
DC = 25
DCH = DPT // (WIN * DC)

ROWS_PER_SUB_DEG = 3 * N // NS
ROWS_PER_SUB_AGG = N // NS

_MESH = plsc.VectorSubcoreMesh(core_axis_name="c", subcore_axis_name="s")
_SC_PARAMS = pltpu.CompilerParams(use_tc_tiling_on_sc=False)


def _deg_call(deg_idx, ones_hbm, zeros_hbm):
  @functools.partial(
      pl.kernel,
      out_type=jax.ShapeDtypeStruct((NC, 2, NS, ROWS_PER_SUB_DEG, 16),
                                    jnp.float32),
      mesh=_MESH,
      scratch_types=[
          pltpu.VMEM((DC, WIN), jnp.int32),
          pltpu.VMEM((WIN, 16), jnp.float32),
          pltpu.VMEM_SHARED((3 * N, 16), jnp.float32),
      ],
      compiler_params=_SC_PARAMS,
  )
  def k(idx_hbm, ones_h, zeros_h, out_hbm, idx_v, ones_v, table_sh):
    cid = lax.axis_index("c")
    sid = lax.axis_index("s")
    wid = sid * NC + cid
    my_rows = pl.ds(sid * ROWS_PER_SUB_DEG, ROWS_PER_SUB_DEG)
    pltpu.sync_copy(ones_h, ones_v)

    for half in range(2):
      pltpu.sync_copy(zeros_h, table_sh.at[my_rows])
      plsc.subcore_barrier()

      @pl.loop(0, DCH)
      def _chunk(ch):
        pltpu.sync_copy(idx_hbm.at[half, wid, ch], idx_v)

        @pl.loop(0, DC)
        def _win(w):
          pltpu.sync_copy(ones_v, table_sh.at[idx_v.at[w]], add=True)

      plsc.subcore_barrier()
      pltpu.sync_copy(table_sh.at[my_rows], out_hbm.at[cid, half, sid])

  return jnp.reshape(k(deg_idx, ones_hbm, zeros_hbm), (NC, 6 * N, 16))


def _h_call(xs, degp):
  bn = 2000
  nb = N // bn

  def body(deg_ref, x_ref, h_ref):
    deg = (deg_ref[0] + deg_ref[1])[:, 0:1]
    norm = jnp.where(deg > 0.0, lax.rsqrt(jnp.maximum(deg, 1.0)), 0.0)
    h_ref[0] = x_ref[0] * norm

  return pl.pallas_call(
      body,
      grid=(3, nb),
      in_specs=[
          pl.BlockSpec((NC, bn, 16), lambda k, j: (0, 2 * k * nb + j, 0)),
          pl.BlockSpec((1, bn, D), lambda k, j: (k, j, 0)),
      ],
      out_specs=pl.BlockSpec((1, bn, D), lambda k, j: (k, j, 0)),
      out_shape=jax.ShapeDtypeStruct((3, N, D), jnp.float32),
  )(degp, xs)


def _msg_call(h, src_idx, dst_idx, zeros_hbm):
  @functools.partial(
      pl.kernel,
      out_type=jax.ShapeDtypeStruct((3, NC, NS, ROWS_PER_SUB_AGG, D),
                                    jnp.float32),
      mesh=_MESH,
      scratch_types=[
          pltpu.VMEM((MW, WIN), jnp.int32),
          pltpu.VMEM((MW, WIN), jnp.int32),
          pltpu.VMEM((WIN, D), jnp.float32),
          pltpu.VMEM((WIN, D), jnp.float32),
          pltpu.VMEM((WIN, D), jnp.float32),
          pltpu.VMEM_SHARED((N, D), jnp.float32),
          pltpu.SemaphoreType.DMA,
          pltpu.SemaphoreType.DMA,
          pltpu.SemaphoreType.DMA,
          pltpu.SemaphoreType.DMA,
          pltpu.SemaphoreType.DMA,
          pltpu.SemaphoreType.DMA,
      ],
      compiler_params=_SC_PARAMS,
  )
  def k(h_hbm, sidx_hbm, didx_hbm, zeros_h, out_hbm, src_v, dst_v, rows_0,
        rows_1, rows_2, agg_sh, gs_0, gs_1, gs_2, ss_0, ss_1, ss_2):
    cid = lax.axis_index("c")
    sid = lax.axis_index("s")
    wid = sid * NC + cid
    my_rows = pl.ds(sid * ROWS_PER_SUB_AGG, ROWS_PER_SUB_AGG)
    rows = [rows_0, rows_1, rows_2]
    gs = [gs_0, gs_1, gs_2]
    ss = [ss_0, ss_1, ss_2]
    NBUF = 3

    for g in range(3):
      pltpu.sync_copy(zeros_h, agg_sh.at[my_rows])
      pltpu.sync_copy(sidx_hbm.at[g, wid], src_v)
      pltpu.sync_copy(didx_hbm.at[g, wid], dst_v)
      plsc.subcore_barrier()

      def g_start(w, j):
        pltpu.async_copy(h_hbm.at[g].at[src_v.at[w]], rows[j], gs[j])

      def g_wait(w, j):
        pltpu.make_async_copy(h_hbm.at[g].at[src_v.at[w]], rows[j],
                              gs[j]).wait()

      def s_start(w, j):
        pltpu.async_copy(rows[j], agg_sh.at[dst_v.at[w]], ss[j], add=True)

      def s_wait(w, j):
        pltpu.make_async_copy(rows[j], agg_sh.at[dst_v.at[w]], ss[j]).wait()

      g_start(0, 0)
      g_start(1, 1)
      g_wait(0, 0); s_start(0, 0); g_start(2, 2)
      s_wait(0, 0); g_wait(1, 1); s_start(1, 1); g_start(3, 0)
      s_wait(1, 1); g_wait(2, 2); s_start(2, 2); g_start(4, 1)

      @pl.loop(1, (MW - 2) // NBUF)
      def _blk(i):
        for j in range(NBUF):
          w = NBUF * i + j
          s_wait(w - 1, (j + 2) % 3)
          g_wait(w, j)
          s_start(w, j)
          g_start(w + 2, (j + 2) % 3)

      s_wait(MW - 3, 2); g_wait(MW - 2, 0); s_start(MW - 2, 0)
      s_wait(MW - 2, 0); g_wait(MW - 1, 1); s_start(MW - 1, 1)
      s_wait(MW - 1, 1)

      plsc.subcore_barrier()
      pltpu.sync_copy(agg_sh.at[my_rows], out_hbm.at[g, cid, sid])
      plsc.subcore_barrier()

  return jnp.reshape(k(h, src_idx, dst_idx, zeros_hbm), (3, NC, N, D))


def _head_call(aggp, degp, W, b):
  bn = 2000
  nb = N // bn
  scale = 1.0 / (3.0 * N * D)

  def body(agg_ref, deg_ref, w_ref, b_ref, out_ref):
    g = pl.program_id(0)
    j = pl.program_id(1)
    deg = (deg_ref[0] + deg_ref[1])[:, 0:1]
    norm = jnp.where(deg > 0.0, lax.rsqrt(jnp.maximum(deg, 1.0)), 0.0)
    agg = (agg_ref[0, 0] + agg_ref[0, 1]) * norm
    y = lax.dot_general(agg, w_ref[...], (((1,), (0,)), ((), ())),
                        preferred_element_type=jnp.float32,
                        precision=lax.Precision.HIGHEST)
    y = jnp.maximum(y + b_ref[...], 0.0)

    @pl.when((g == 0) & (j == 0))
    def _():
      out_ref[...] = jnp.zeros((1, 1), jnp.float32)

    out_ref[...] += jnp.reshape(jnp.sum(y) * scale, (1, 1))

  return pl.pallas_call(
      body,
      grid=(3, nb),
      in_specs=[
          pl.BlockSpec((1, NC, bn, D), lambda k, j: (k, 0, j, 0)),
          pl.BlockSpec((NC, bn, 16), lambda k, j: (0, (2 * k + 1) * nb + j, 0)),
          pl.BlockSpec((D, D), lambda k, j: (0, 0)),
          pl.BlockSpec((1, D), lambda k, j: (0, 0)),
      ],
      out_specs=pl.BlockSpec((1, 1), lambda k, j: (0, 0)),
      out_shape=jax.ShapeDtypeStruct((1, 1), jnp.float32),
  )(aggp, degp, W, b)


@jax.jit
def kernel(g1, x1, g2, x2, g3, x3, W, b):
  xs = jnp.stack([x1, x2, x3])
  deg_idx = jnp.stack([
      jnp.concatenate([g1[0], g1[1] + N, g2[0] + 2 * N]),
      jnp.concatenate([g2[1], g3[0] + N, g3[1] + 2 * N]),
  ]).reshape(2, NW, DCH, DC, WIN)
  src_idx = jnp.stack([g1[0], g2[0], g3[0]]).reshape(3, NW, MW, WIN)
  dst_idx = jnp.stack([g1[1], g2[1], g3[1]]).reshape(3, NW, MW, WIN)
  ones16 = jnp.ones((WIN, 16), jnp.float32)
  zeros_deg = jnp.zeros((ROWS_PER_SUB_DEG, 16), jnp.float32)
  zeros_agg = jnp.zeros((ROWS_PER_SUB_AGG, D), jnp.float32)

  degp = _deg_call(deg_idx, ones16, zeros_deg)
  h = _h_call(xs, degp)
  aggp = _msg_call(h, src_idx, dst_idx, zeros_agg)
  out = _head_call(aggp, degp, W, jnp.reshape(b, (1, D)))
  return out[0, 0]


# speedup vs baseline: 10.5060x; 1.1187x over previous
import functools

import jax
import jax.numpy as jnp
from jax import lax
from jax.experimental import pallas as pl
from jax.experimental.pallas import tpu as pltpu
from jax.experimental.pallas import tpu_sc as plsc

N = 10000
E = 320000
D = 128

NC = 2
NS = 16
NW = NC * NS

WIN = 80
MW = E // (NW * WIN)
DPT = 3 * E // NW
DWIN = 120
DC = 25
DCH = DPT // (DWIN * DC)

ROWS_PER_SUB_DEG = 3 * N // NS
ROWS_PER_SUB_AGG = N // NS

_MESH = plsc.VectorSubcoreMesh(core_axis_name="c", subcore_axis_name="s")
_SC_PARAMS = pltpu.CompilerParams(use_tc_tiling_on_sc=False)


def _deg_call(deg_idx, ones_hbm, zeros_hbm):
  @functools.partial(
      pl.kernel,
      out_type=jax.ShapeDtypeStruct((NC, 2, NS, ROWS_PER_SUB_DEG, 16),
                                    jnp.float32),
      mesh=_MESH,
      scratch_types=[
          pltpu.VMEM((DC, DWIN), jnp.int32),
          pltpu.VMEM((DC, DWIN), jnp.int32),
          pltpu.VMEM((DWIN, 16), jnp.float32),
          pltpu.VMEM_SHARED((3 * N, 16), jnp.float32),
          pltpu.SemaphoreType.DMA,
          pltpu.SemaphoreType.DMA,
          pltpu.SemaphoreType.DMA,
          pltpu.SemaphoreType.DMA,
      ],
      compiler_params=_SC_PARAMS,
  )
  def k(idx_hbm, ones_h, zeros_h, out_hbm, idx_a, idx_b, ones_v, table_sh,
        la, lb, sa, sb):
    cid = lax.axis_index("c")
    sid = lax.axis_index("s")
    wid = sid * NC + cid
    my_rows = pl.ds(sid * ROWS_PER_SUB_DEG, ROWS_PER_SUB_DEG)
    pltpu.sync_copy(ones_h, ones_v)
    idx = [idx_a, idx_b]
    lsem = [la, lb]
    ssem = [sa, sb]

    def load_start(half, c, x):
      pltpu.async_copy(idx_hbm.at[half, wid, c], idx[x], lsem[x])

    def load_wait(half, c, x):
      pltpu.make_async_copy(idx_hbm.at[half, wid, c], idx[x], lsem[x]).wait()

    def fire(x):
      @pl.loop(0, DC)
      def _win(w):
        pltpu.async_copy(ones_v, table_sh.at[idx[x].at[w]], ssem[x], add=True)

    def drain(x):
      @pl.loop(0, DC)
      def _win(w):
        pltpu.make_async_copy(ones_v, table_sh.at[idx[x].at[w]],
                              ssem[x]).wait()

    for half in range(2):
      pltpu.sync_copy(zeros_h, table_sh.at[my_rows])
      plsc.subcore_barrier()

      load_start(half, 0, 0)
      load_wait(half, 0, 0)
      fire(0)
      load_start(half, 1, 1)

      @pl.loop(0, (DCH - 2) // 2)
      def _pair(i):
        c = 2 * i + 1
        load_wait(half, c, 1)
        fire(1)
        drain(0)
        load_start(half, c + 1, 0)
        load_wait(half, c + 1, 0)
        fire(0)
        drain(1)
        load_start(half, c + 2, 1)

      load_wait(half, DCH - 1, 1)
      fire(1)
      drain(0)
      drain(1)

      plsc.subcore_barrier()
      pltpu.sync_copy(table_sh.at[my_rows], out_hbm.at[cid, half, sid])

  return jnp.reshape(k(deg_idx, ones_hbm, zeros_hbm), (NC, 6 * N, 16))


def _h_call(xs, degp):
  bn = 2000
  nb = N // bn

  def body(deg_ref, x_ref, h_ref):
    deg = (deg_ref[0] + deg_ref[1])[:, 0:1]
    norm = jnp.where(deg > 0.0, lax.rsqrt(jnp.maximum(deg, 1.0)), 0.0)
    h_ref[0] = x_ref[0] * norm

  return pl.pallas_call(
      body,
      grid=(3, nb),
      in_specs=[
          pl.BlockSpec((NC, bn, 16), lambda k, j: (0, 2 * k * nb + j, 0)),
          pl.BlockSpec((1, bn, D), lambda k, j: (k, j, 0)),
      ],
      out_specs=pl.BlockSpec((1, bn, D), lambda k, j: (k, j, 0)),
      out_shape=jax.ShapeDtypeStruct((3, N, D), jnp.float32),
  )(degp, xs)


def _msg_call(h, src_idx, dst_idx, zeros_hbm):
  @functools.partial(
      pl.kernel,
      out_type=jax.ShapeDtypeStruct((3, NC, NS, ROWS_PER_SUB_AGG, D),
                                    jnp.float32),
      mesh=_MESH,
      scratch_types=[
          pltpu.VMEM((MW, WIN), jnp.int32),
          pltpu.VMEM((MW, WIN), jnp.int32),
          pltpu.VMEM((WIN, D), jnp.float32),
          pltpu.VMEM((WIN, D), jnp.float32),
          pltpu.VMEM((WIN, D), jnp.float32),
          pltpu.VMEM_SHARED((N, D), jnp.float32),
          pltpu.SemaphoreType.DMA,
          pltpu.SemaphoreType.DMA,
          pltpu.SemaphoreType.DMA,
          pltpu.SemaphoreType.DMA,
          pltpu.SemaphoreType.DMA,
          pltpu.SemaphoreType.DMA,
      ],
      compiler_params=_SC_PARAMS,
  )
  def k(h_hbm, sidx_hbm, didx_hbm, zeros_h, out_hbm, src_v, dst_v, rows_0,
        rows_1, rows_2, agg_sh, gs_0, gs_1, gs_2, ss_0, ss_1, ss_2):
    cid = lax.axis_index("c")
    sid = lax.axis_index("s")
    wid = sid * NC + cid
    my_rows = pl.ds(sid * ROWS_PER_SUB_AGG, ROWS_PER_SUB_AGG)
    rows = [rows_0, rows_1, rows_2]
    gs = [gs_0, gs_1, gs_2]
    ss = [ss_0, ss_1, ss_2]
    NBUF = 3

    for g in range(3):
      pltpu.sync_copy(zeros_h, agg_sh.at[my_rows])
      pltpu.sync_copy(sidx_hbm.at[g, wid], src_v)
      pltpu.sync_copy(didx_hbm.at[g, wid], dst_v)
      plsc.subcore_barrier()

      def g_start(w, j):
        pltpu.async_copy(h_hbm.at[g].at[src_v.at[w]], rows[j], gs[j])

      def g_wait(w, j):
        pltpu.make_async_copy(h_hbm.at[g].at[src_v.at[w]], rows[j],
                              gs[j]).wait()

      def s_start(w, j):
        pltpu.async_copy(rows[j], agg_sh.at[dst_v.at[w]], ss[j], add=True)

      def s_wait(w, j):
        pltpu.make_async_copy(rows[j], agg_sh.at[dst_v.at[w]], ss[j]).wait()

      g_start(0, 0)
      g_start(1, 1)
      g_wait(0, 0); s_start(0, 0); g_start(2, 2)
      s_wait(0, 0); g_wait(1, 1); s_start(1, 1); g_start(3, 0)
      s_wait(1, 1); g_wait(2, 2); s_start(2, 2); g_start(4, 1)

      @pl.loop(1, (MW - 2) // NBUF)
      def _blk(i):
        for j in range(NBUF):
          w = NBUF * i + j
          s_wait(w - 1, (j + 2) % 3)
          g_wait(w, j)
          s_start(w, j)
          g_start(w + 2, (j + 2) % 3)

      s_wait(MW - 3, 2); g_wait(MW - 2, 0); s_start(MW - 2, 0)
      s_wait(MW - 2, 0); g_wait(MW - 1, 1); s_start(MW - 1, 1)
      s_wait(MW - 1, 1)

      plsc.subcore_barrier()
      pltpu.sync_copy(agg_sh.at[my_rows], out_hbm.at[g, cid, sid])
      plsc.subcore_barrier()

  return jnp.reshape(k(h, src_idx, dst_idx, zeros_hbm), (3, NC, N, D))


def _head_call(aggp, degp, W, b):
  bn = 2000
  nb = N // bn
  scale = 1.0 / (3.0 * N * D)

  def body(agg_ref, deg_ref, w_ref, b_ref, out_ref):
    g = pl.program_id(0)
    j = pl.program_id(1)
    deg = (deg_ref[0] + deg_ref[1])[:, 0:1]
    norm = jnp.where(deg > 0.0, lax.rsqrt(jnp.maximum(deg, 1.0)), 0.0)
    agg = (agg_ref[0, 0] + agg_ref[0, 1]) * norm
    y = lax.dot_general(agg, w_ref[...], (((1,), (0,)), ((), ())),
                        preferred_element_type=jnp.float32,
                        precision=lax.Precision.HIGHEST)
    y = jnp.maximum(y + b_ref[...], 0.0)

    @pl.when((g == 0) & (j == 0))
    def _():
      out_ref[...] = jnp.zeros((1, 1), jnp.float32)

    out_ref[...] += jnp.reshape(jnp.sum(y) * scale, (1, 1))

  return pl.pallas_call(
      body,
      grid=(3, nb),
      in_specs=[
          pl.BlockSpec((1, NC, bn, D), lambda k, j: (k, 0, j, 0)),
          pl.BlockSpec((NC, bn, 16), lambda k, j: (0, (2 * k + 1) * nb + j, 0)),
          pl.BlockSpec((D, D), lambda k, j: (0, 0)),
          pl.BlockSpec((1, D), lambda k, j: (0, 0)),
      ],
      out_specs=pl.BlockSpec((1, 1), lambda k, j: (0, 0)),
      out_shape=jax.ShapeDtypeStruct((1, 1), jnp.float32),
  )(aggp, degp, W, b)


@jax.jit
def kernel(g1, x1, g2, x2, g3, x3, W, b):
  xs = jnp.stack([x1, x2, x3])
  deg_idx = jnp.stack([
      jnp.concatenate([g1[0], g1[1] + N, g2[0] + 2 * N]),
      jnp.concatenate([g2[1], g3[0] + N, g3[1] + 2 * N]),
  ]).reshape(2, NW, DCH, DC, DWIN)
  src_idx = jnp.stack([g1[0], g2[0], g3[0]]).reshape(3, NW, MW, WIN)
  dst_idx = jnp.stack([g1[1], g2[1], g3[1]]).reshape(3, NW, MW, WIN)
  ones16 = jnp.ones((DWIN, 16), jnp.float32)
  zeros_deg = jnp.zeros((ROWS_PER_SUB_DEG, 16), jnp.float32)
  zeros_agg = jnp.zeros((ROWS_PER_SUB_AGG, D), jnp.float32)

  degp = _deg_call(deg_idx, ones16, zeros_deg)
  h = _h_call(xs, degp)
  aggp = _msg_call(h, src_idx, dst_idx, zeros_agg)
  out = _head_call(aggp, degp, W, jnp.reshape(b, (1, D)))
  return out[0, 0]
